# baseline (speedup 1.0000x reference)
import functools

import jax
import jax.numpy as jnp
from jax.experimental import pallas as pl
from jax.experimental.pallas import tpu as pltpu

NUM_HEADS = 12
DIM = 768
HEAD_DIM = DIM // NUM_HEADS
BQ = 512


def _body(x_full_ref, x_blk_ref, wq_ref, wk_ref, wv_ref,
          bq_ref, bk_ref, bv_ref, wp_ref, bp_ref,
          out_ref, k_scr, v_scr, *, scale):
    i = pl.program_id(0)
    h = pl.program_id(1)

    bf = jnp.bfloat16

    @pl.when(i == 0)
    def _():
        xf = x_full_ref[...].astype(bf)
        k_scr[h] = (jnp.dot(xf, wk_ref[0].astype(bf),
                            preferred_element_type=jnp.float32)
                    + bk_ref[0]).astype(bf)
        v_scr[h] = (jnp.dot(xf, wv_ref[0].astype(bf),
                            preferred_element_type=jnp.float32)
                    + bv_ref[0]).astype(bf)

    q = (jnp.dot(x_blk_ref[...].astype(bf), wq_ref[0].astype(bf),
                 preferred_element_type=jnp.float32)
         + bq_ref[0]) * scale
    s = jax.lax.dot_general(q.astype(bf), k_scr[h], (((1,), (1,)), ((), ())),
                            preferred_element_type=jnp.float32)
    s = s - jnp.max(s, axis=-1, keepdims=True)
    p = jnp.exp(s)
    o = jnp.dot(p.astype(bf), v_scr[h], preferred_element_type=jnp.float32)
    o = o / jnp.sum(p, axis=-1, keepdims=True)
    contrib = jnp.dot(o.astype(bf), wp_ref[...].astype(bf),
                      preferred_element_type=jnp.float32)

    @pl.when(h == 0)
    def _():
        out_ref[...] = contrib + bp_ref[...]

    @pl.when(h > 0)
    def _():
        out_ref[...] += contrib


@jax.jit
def kernel(x, W_qkv, b_qkv, W_proj, b_proj):
    B, N, C = x.shape
    H, D = NUM_HEADS, HEAD_DIM
    scale = D ** -0.5
    x2 = x.reshape(N, C)
    W = W_qkv.reshape(C, 3, H, D)
    Wq = W[:, 0].transpose(1, 0, 2)
    Wk = W[:, 1].transpose(1, 0, 2)
    Wv = W[:, 2].transpose(1, 0, 2)
    b3 = b_qkv.reshape(3, H, 1, D)
    bq, bk, bv = b3[0], b3[1], b3[2]
    bp = b_proj.reshape(1, C)

    nq = N // BQ
    out = pl.pallas_call(
        functools.partial(_body, scale=scale),
        grid=(nq, H),
        in_specs=[
            pl.BlockSpec((N, C), lambda i, h: (0, 0)),
            pl.BlockSpec((BQ, C), lambda i, h: (i, 0)),
            pl.BlockSpec((1, C, D), lambda i, h: (h, 0, 0)),
            pl.BlockSpec((1, C, D), lambda i, h: (h, 0, 0)),
            pl.BlockSpec((1, C, D), lambda i, h: (h, 0, 0)),
            pl.BlockSpec((1, 1, D), lambda i, h: (h, 0, 0)),
            pl.BlockSpec((1, 1, D), lambda i, h: (h, 0, 0)),
            pl.BlockSpec((1, 1, D), lambda i, h: (h, 0, 0)),
            pl.BlockSpec((D, C), lambda i, h: (h, 0)),
            pl.BlockSpec((1, C), lambda i, h: (0, 0)),
        ],
        out_specs=pl.BlockSpec((BQ, C), lambda i, h: (i, 0)),
        out_shape=jax.ShapeDtypeStruct((N, C), jnp.float32),
        scratch_shapes=[
            pltpu.VMEM((H, N, D), jnp.bfloat16),
            pltpu.VMEM((H, N, D), jnp.bfloat16),
        ],
        compiler_params=pltpu.CompilerParams(
            dimension_semantics=("arbitrary", "arbitrary"),
        ),
    )(x2, x2, Wq, Wk, Wv, bq, bk, bv, W_proj, bp)
    return out.reshape(B, N, C)


# speedup vs baseline: 1.7998x; 1.7998x over previous
import functools

import jax
import jax.numpy as jnp
from jax.experimental import pallas as pl
from jax.experimental.pallas import tpu as pltpu

NUM_HEADS = 12
DIM = 768
HEAD_DIM = DIM // NUM_HEADS
BQ = 512
PAIR = 2 * HEAD_DIM


def _attend(q, k, v):
    s = jax.lax.dot_general(q, k, (((1,), (1,)), ((), ())),
                            preferred_element_type=jnp.float32)
    s = s - jnp.max(s, axis=-1, keepdims=True)
    p = jnp.exp(s)
    o = jnp.dot(p, v, preferred_element_type=jnp.float32)
    return o / jnp.sum(p, axis=-1, keepdims=True)


def _body(x_full_ref, x_blk_ref, wq_ref, wk_ref, wv_ref,
          bq_ref, bk_ref, bv_ref, wp_ref, bp_ref,
          out_ref, k_scr, v_scr, *, scale):
    i = pl.program_id(0)
    j = pl.program_id(1)
    D = HEAD_DIM

    @pl.when(i == 0)
    def _():
        xf = x_full_ref[...]
        k_scr[j] = (jnp.dot(xf, wk_ref[...], preferred_element_type=jnp.float32)
                    + bk_ref[0])
        v_scr[j] = (jnp.dot(xf, wv_ref[...], preferred_element_type=jnp.float32)
                    + bv_ref[0])

    qq = (jnp.dot(x_blk_ref[...], wq_ref[...], preferred_element_type=jnp.float32)
          + bq_ref[0]) * scale
    kk = k_scr[j]
    vv = v_scr[j]
    o_a = _attend(qq[:, :D], kk[:, :D], vv[:, :D])
    o_b = _attend(qq[:, D:], kk[:, D:], vv[:, D:])
    o = jnp.concatenate([o_a, o_b], axis=1)
    contrib = jnp.dot(o, wp_ref[...], preferred_element_type=jnp.float32)

    @pl.when(j == 0)
    def _():
        out_ref[...] = contrib + bp_ref[...]

    @pl.when(j > 0)
    def _():
        out_ref[...] += contrib


@jax.jit
def kernel(x, W_qkv, b_qkv, W_proj, b_proj):
    B, N, C = x.shape
    H, D = NUM_HEADS, HEAD_DIM
    NP = H // 2
    scale = D ** -0.5
    x2 = x.reshape(N, C)
    b_qkv3 = b_qkv.reshape(3 * NP, 1, PAIR)
    bp = b_proj.reshape(1, C)

    nq = N // BQ
    out = pl.pallas_call(
        functools.partial(_body, scale=scale),
        grid=(nq, NP),
        in_specs=[
            pl.BlockSpec((N, C), lambda i, j: (0, 0)),
            pl.BlockSpec((BQ, C), lambda i, j: (i, 0)),
            pl.BlockSpec((C, PAIR), lambda i, j: (0, j)),
            pl.BlockSpec((C, PAIR), lambda i, j: (0, NP + j)),
            pl.BlockSpec((C, PAIR), lambda i, j: (0, 2 * NP + j)),
            pl.BlockSpec((1, 1, PAIR), lambda i, j: (j, 0, 0)),
            pl.BlockSpec((1, 1, PAIR), lambda i, j: (NP + j, 0, 0)),
            pl.BlockSpec((1, 1, PAIR), lambda i, j: (2 * NP + j, 0, 0)),
            pl.BlockSpec((PAIR, C), lambda i, j: (j, 0)),
            pl.BlockSpec((1, C), lambda i, j: (0, 0)),
        ],
        out_specs=pl.BlockSpec((BQ, C), lambda i, j: (i, 0)),
        out_shape=jax.ShapeDtypeStruct((N, C), jnp.float32),
        scratch_shapes=[
            pltpu.VMEM((NP, N, PAIR), jnp.float32),
            pltpu.VMEM((NP, N, PAIR), jnp.float32),
        ],
        compiler_params=pltpu.CompilerParams(
            dimension_semantics=("arbitrary", "arbitrary"),
        ),
    )(x2, x2, W_qkv, W_qkv, W_qkv, b_qkv3, b_qkv3, b_qkv3, W_proj, bp)
    return out.reshape(B, N, C)
